# baseline (speedup 1.0000x reference)
import functools

import jax
import jax.numpy as jnp
from jax import lax
from jax.experimental import pallas as pl
from jax.experimental.pallas import tpu as pltpu

_DN_T = (((1,), (1,)), ((), ()))


def _cell_block(x_ref, w0_ref, w2_ref, wfc_ref, b0_ref, g1_ref, c1_ref,
                g2_ref, c2_ref, b2_ref, bfc_ref, out_ref):
    inv_std = 1.0 / jnp.sqrt(1.0 + 1e-5)
    x = x_ref[...]
    h1 = lax.dot_general(x, w0_ref[...], _DN_T,
                         preferred_element_type=jnp.float32)
    h1 = h1 + b0_ref[...]
    t1 = jnp.maximum(h1 * (inv_std * g1_ref[...]) + c1_ref[...], 0.0)
    t2 = jnp.maximum(h1 * (inv_std * g2_ref[...]) + c2_ref[...], 0.0)
    h3 = lax.dot_general(t2, w2_ref[...], _DN_T,
                         preferred_element_type=jnp.float32)
    h3 = jnp.maximum(h3 + b2_ref[...], 0.0)
    cat = jnp.concatenate([t1, h3], axis=1)
    acc = lax.dot_general(cat, wfc_ref[...], _DN_T,
                          preferred_element_type=jnp.float32)
    out_ref[...] = acc + bfc_ref[...]


@functools.partial(jax.jit, static_argnames=("block_n",))
def _cell(x, W0, b0, W2, b2, bn1_g, bn1_b, bn2_g, bn2_b, Wfc, bfc,
          block_n=1000):
    n, d = x.shape
    grid = (n // block_n,)
    row_spec = pl.BlockSpec((block_n, d), lambda i: (i, 0))
    full = lambda shape: pl.BlockSpec(shape, lambda i: (0,) * len(shape))

    return pl.pallas_call(
        _cell_block,
        grid=grid,
        in_specs=[
            row_spec,
            full((d, d)), full((d, d)), full((d, 2 * d)),
            full((1, d)), full((1, d)), full((1, d)), full((1, d)),
            full((1, d)), full((1, d)), full((1, d)),
        ],
        out_specs=row_spec,
        out_shape=jax.ShapeDtypeStruct((n, d), jnp.float32),
        compiler_params=pltpu.CompilerParams(
            dimension_semantics=("arbitrary",)),
    )(x, W0, W2, Wfc, b0.reshape(1, d), bn1_g.reshape(1, d),
      bn1_b.reshape(1, d), bn2_g.reshape(1, d), bn2_b.reshape(1, d),
      b2.reshape(1, d), bfc.reshape(1, d))


def kernel(x, edge_index, W0, b0, W2, b2, bn1_g, bn1_b, bn2_g, bn2_b, Wfc, bfc):
    del edge_index
    return _cell(x, W0, b0, W2, b2, bn1_g, bn1_b, bn2_g, bn2_b, Wfc, bfc)


# speedup vs baseline: 1.4284x; 1.4284x over previous
import functools

import jax
import jax.numpy as jnp
from jax import lax
from jax.experimental import pallas as pl
from jax.experimental.pallas import tpu as pltpu

_DN_T = (((1,), (1,)), ((), ()))


def _cell_block(x_ref, w0_ref, w2_ref, wfc_ref, b0_ref, g1_ref, c1_ref,
                g2_ref, c2_ref, b2_ref, bfc_ref, out_ref):
    inv_std = 1.0 / jnp.sqrt(1.0 + 1e-5)
    x = x_ref[...]
    h1 = lax.dot_general(x, w0_ref[...], _DN_T,
                         preferred_element_type=jnp.float32)
    h1 = h1 + b0_ref[...]
    t1 = jnp.maximum(h1 * (inv_std * g1_ref[...]) + c1_ref[...], 0.0)
    t2 = jnp.maximum(h1 * (inv_std * g2_ref[...]) + c2_ref[...], 0.0)
    h3 = lax.dot_general(t2, w2_ref[...], _DN_T,
                         preferred_element_type=jnp.float32)
    h3 = jnp.maximum(h3 + b2_ref[...], 0.0)
    cat = jnp.concatenate([t1, h3], axis=1)
    acc = lax.dot_general(cat, wfc_ref[...], _DN_T,
                          preferred_element_type=jnp.float32)
    out_ref[...] = acc + bfc_ref[...]


@functools.partial(jax.jit, static_argnames=("block_n",))
def _cell(x, W0, b0, W2, b2, bn1_g, bn1_b, bn2_g, bn2_b, Wfc, bfc,
          block_n=5000):
    n, d = x.shape
    grid = (n // block_n,)
    row_spec = pl.BlockSpec((block_n, d), lambda i: (i, 0))
    full = lambda shape: pl.BlockSpec(shape, lambda i: (0,) * len(shape))

    return pl.pallas_call(
        _cell_block,
        grid=grid,
        in_specs=[
            row_spec,
            full((d, d)), full((d, d)), full((d, 2 * d)),
            full((1, d)), full((1, d)), full((1, d)), full((1, d)),
            full((1, d)), full((1, d)), full((1, d)),
        ],
        out_specs=row_spec,
        out_shape=jax.ShapeDtypeStruct((n, d), jnp.float32),
        compiler_params=pltpu.CompilerParams(
            dimension_semantics=("arbitrary",)),
    )(x, W0, W2, Wfc, b0.reshape(1, d), bn1_g.reshape(1, d),
      bn1_b.reshape(1, d), bn2_g.reshape(1, d), bn2_b.reshape(1, d),
      b2.reshape(1, d), bfc.reshape(1, d))


def kernel(x, edge_index, W0, b0, W2, b2, bn1_g, bn1_b, bn2_g, bn2_b, Wfc, bfc):
    del edge_index
    return _cell(x, W0, b0, W2, b2, bn1_g, bn1_b, bn2_g, bn2_b, Wfc, bfc)
